# Initial kernel scaffold; baseline (speedup 1.0000x reference)
import jax
import jax.numpy as jnp
from jax.experimental import pallas as pl


def kernel(h_V, h_E, E_idx, W1, b1, W2, b2, W3, b3, W11, b11, W12, b12, W13, b13, Wi, bi, Wo, bo, g1, be1, g2, be2, g3, be3):
    raise NotImplementedError("write your pallas kernel here")


# speedup vs baseline: 3.8725x; 3.8725x over previous
import functools

import jax
import jax.numpy as jnp
from jax import lax
from jax.experimental import pallas as pl
from jax.experimental.pallas import tpu as pltpu
from jax.experimental.pallas import tpu_sc as plsc

B, N, K, H, FF = 1, 10000, 32, 128, 512
R = N * K
SCALE = 30.0
EPS = 1e-5

_NC, _NS = 2, 16
_NW = _NC * _NS
_PW = R // _NW
_CH = 80
_NCH = _PW // _CH


def _sc_gather(table, idx):
    mesh = plsc.VectorSubcoreMesh(core_axis_name="c", subcore_axis_name="s")

    @functools.partial(
        pl.kernel,
        out_type=jax.ShapeDtypeStruct((R, H), jnp.float32),
        mesh=mesh,
        scratch_types=[
            pltpu.VMEM((_CH,), jnp.int32),
            pltpu.VMEM((_CH, H), jnp.float32),
            pltpu.SemaphoreType.DMA,
        ],
    )
    def gk(table_hbm, idx_hbm, out_hbm, idx_v, rows_v, sem):
        wid = lax.axis_index("s") * _NC + lax.axis_index("c")
        base = wid * _PW

        def body(c, carry):
            off = base + c * _CH
            pltpu.sync_copy(idx_hbm.at[pl.ds(off, _CH)], idx_v)
            pltpu.async_copy(table_hbm.at[idx_v], rows_v, sem).wait()
            pltpu.sync_copy(rows_v, out_hbm.at[pl.ds(off, _CH)])
            return carry

        lax.fori_loop(0, _NCH, body, 0)

    return gk(table, idx)


_TN = 200


def _gelu(x):
    return 0.5 * x * (1.0 + lax.erf(x * 0.7071067811865476))


def _ln(x, g, b):
    m = jnp.mean(x, axis=-1, keepdims=True)
    xc = x - m
    v = jnp.mean(xc * xc, axis=-1, keepdims=True)
    return xc * lax.rsqrt(v + EPS) * g + b


def _dot(a, b):
    return jnp.dot(a, b, preferred_element_type=jnp.float32)


def _stage_a_body(hv_ref, he_ref, gg_ref, w1a, w1b, w1c, b1r, w2, b2r, w3,
                  b3r, wi, bir, wo, bor, g1r, be1r, g2r, be2r, out_ref):
    hv = hv_ref[...]
    he = he_ref[...]
    gg = gg_ref[...]
    tv = _dot(hv, w1a[...])
    z = _dot(he, w1b[...]) + _dot(gg, w1c[...]) + b1r[...]
    z = z.reshape(_TN, K, H) + tv[:, None, :]
    m1 = _gelu(z).reshape(_TN * K, H)
    m2 = _gelu(_dot(m1, w2[...]) + b2r[...])
    m3 = _dot(m2, w3[...]) + b3r[...]
    dh = jnp.sum(m3.reshape(_TN, K, H), axis=1) * (1.0 / SCALE)
    x = _ln(hv + dh, g1r[...], be1r[...])
    f = _gelu(_dot(x, wi[...]) + bir[...])
    x2 = x + _dot(f, wo[...]) + bor[...]
    out_ref[...] = _ln(x2, g2r[...], be2r[...])


def _stage_c_body(hv_ref, he_ref, gg_ref, w1a, w1b, w1c, b1r, w2, b2r, w3,
                  b3r, g3r, be3r, out_ref):
    hv = hv_ref[...]
    he = he_ref[...]
    gg = gg_ref[...]
    tv = _dot(hv, w1a[...])
    z = _dot(he, w1b[...]) + _dot(gg, w1c[...]) + b1r[...]
    z = z.reshape(_TN, K, H) + tv[:, None, :]
    m1 = _gelu(z).reshape(_TN * K, H)
    m2 = _gelu(_dot(m1, w2[...]) + b2r[...])
    m3 = _dot(m2, w3[...]) + b3r[...]
    out_ref[...] = _ln(he + m3, g3r[...], be3r[...])


def _node_spec():
    return pl.BlockSpec((_TN, H), lambda i: (i, 0))


def _edge_spec():
    return pl.BlockSpec((_TN * K, H), lambda i: (i, 0))


def _w_spec(r, c):
    return pl.BlockSpec((r, c), lambda i: (0, 0))


def _stage_a(hv, he2, gg, w1a, w1b, w1c, b1, w2, b2, w3, b3, wi, bi, wo, bo,
             g1, be1, g2, be2):
    grid = (N // _TN,)
    in_specs = [
        _node_spec(), _edge_spec(), _edge_spec(),
        _w_spec(H, H), _w_spec(H, H), _w_spec(H, H), _w_spec(1, H),
        _w_spec(H, H), _w_spec(1, H), _w_spec(H, H), _w_spec(1, H),
        _w_spec(H, FF), _w_spec(1, FF), _w_spec(FF, H), _w_spec(1, H),
        _w_spec(1, H), _w_spec(1, H), _w_spec(1, H), _w_spec(1, H),
    ]
    return pl.pallas_call(
        _stage_a_body,
        grid=grid,
        in_specs=in_specs,
        out_specs=_node_spec(),
        out_shape=jax.ShapeDtypeStruct((N, H), jnp.float32),
    )(hv, he2, gg, w1a, w1b, w1c, b1, w2, b2, w3, b3, wi, bi, wo, bo,
      g1, be1, g2, be2)


def _stage_c(hv2, he2, gg, w1a, w1b, w1c, b1, w2, b2, w3, b3, g3, be3):
    grid = (N // _TN,)
    in_specs = [
        _node_spec(), _edge_spec(), _edge_spec(),
        _w_spec(H, H), _w_spec(H, H), _w_spec(H, H), _w_spec(1, H),
        _w_spec(H, H), _w_spec(1, H), _w_spec(H, H), _w_spec(1, H),
        _w_spec(1, H), _w_spec(1, H),
    ]
    return pl.pallas_call(
        _stage_c_body,
        grid=grid,
        in_specs=in_specs,
        out_specs=_edge_spec(),
        out_shape=jax.ShapeDtypeStruct((R, H), jnp.float32),
    )(hv2, he2, gg, w1a, w1b, w1c, b1, w2, b2, w3, b3, g3, be3)


def kernel(h_V, h_E, E_idx, W1, b1, W2, b2, W3, b3, W11, b11, W12, b12,
           W13, b13, Wi, bi, Wo, bo, g1, be1, g2, be2, g3, be3):
    hv = h_V[0]
    he2 = h_E[0].reshape(R, H)
    idx = E_idx[0].reshape(R)

    row = lambda v: v.reshape(1, -1)

    g1v = _sc_gather(hv, idx)
    hv2 = _stage_a(hv, he2, g1v,
                   W1[:H], W1[H:2 * H], W1[2 * H:], row(b1),
                   W2, row(b2), W3, row(b3),
                   Wi, row(bi), Wo, row(bo),
                   row(g1), row(be1), row(g2), row(be2))
    g2v = _sc_gather(hv2, idx)
    he_out = _stage_c(hv2, he2, g2v,
                      W11[:H], W11[H:2 * H], W11[2 * H:], row(b11),
                      W12, row(b12), W13, row(b13),
                      row(g3), row(be3))
    return hv2[None], he_out.reshape(B, N, K, H)
